# baseline (speedup 1.0000x reference)
import functools
import jax
import jax.numpy as jnp
from jax.experimental import pallas as pl
from jax.experimental.pallas import tpu as pltpu

C = 16
RELAX_DENOM = 2.0
NUM_OFF = C * (C - 1) / 2.0
MARGIN = float(int(NUM_OFF // RELAX_DENOM))

NCH = 128
MROW = 256
KSTEPS = MROW // NCH
SSTR = NCH + 1
TSTEPS = 3 * KSTEPS


def _x_imap(i, b, t, chan_ref):
    tf = jnp.minimum(t, 2 * KSTEPS - 1)
    return (b, chan_ref[(tf // KSTEPS) * MROW + (tf % KSTEPS) * NCH + i], 0, 0)


def _sd_imap(b, t, chan_ref):
    return (jnp.clip((t - KSTEPS) // KSTEPS, 0, 1), 0, 0)


def _out_imap(b, t, chan_ref):
    return (b, jnp.clip((t - KSTEPS) // KSTEPS, 0, 1), 0, 0)


def _saw_body(chan_ref, *refs):
    xrefs = refs[:NCH]
    mask_ref = refs[NCH]
    out_ref = refs[NCH + 1]
    tile = refs[NCH + 2]
    acc = refs[NCH + 3]
    t = pl.program_id(1)
    nh = xrefs[0].shape[2]

    @pl.when(t < 2 * KSTEPS)
    def _():
        buf = (t // KSTEPS) % 2
        tk = tile.at[buf, t % KSTEPS]
        for i in range(NCH):
            tk[i : i + SSTR * nh : SSTR, :] = xrefs[i][0, 0]

    @pl.when(t >= KSTEPS)
    def _():
        jt = t - KSTEPS
        dbuf = (jt // KSTEPS) % 2
        ndots = nh // 2
        per = ndots // KSTEPS

        def dot_phase(p):
            @pl.when(jt % KSTEPS == p)
            def _():
                covs = []
                for j in range(p * per, (p + 1) * per):
                    lhs = jnp.concatenate(
                        [
                            jnp.concatenate(
                                [tile[dbuf, kk, jj * SSTR : jj * SSTR + NCH, :]
                                 for kk in range(KSTEPS)],
                                axis=0,
                            )
                            for jj in (2 * j, 2 * j + 1)
                        ],
                        axis=1,
                    )
                    covs.append(
                        jax.lax.dot_general(
                            lhs, lhs.T, (((1,), (0,)), ((), ())),
                            preferred_element_type=jnp.float32,
                        )
                    )
                while len(covs) > 1:
                    covs = [a + b for a, b in zip(covs[::2], covs[1::2])] + (
                        [covs[-1]] if len(covs) % 2 else [])
                part = covs[0]
                if p == 0 and KSTEPS > 1:
                    acc[...] = part
                if p == KSTEPS - 1:
                    cov = part if KSTEPS == 1 else acc[...] + part
                    tm = jnp.abs(cov) * mask_ref[0]
                    rs = jnp.sum(tm, axis=1, keepdims=True)
                    gs = jnp.sum(rs.reshape(C, C, 1), axis=1)
                    out_ref[0, 0] = jnp.maximum((gs - MARGIN) / NUM_OFF, 0.0)

        for p in range(KSTEPS):
            dot_phase(p)


def kernel(x, classifier_weight):
    B, ch, H, W = x.shape
    G = ch // C
    nstack = ch // MROW
    hw = H * W
    w = jnp.abs(classifier_weight)
    idx = jnp.argsort(-w, axis=1)
    idx_sel = idx[:C, :G]
    sig = jax.nn.sigmoid(w)[:C]
    chan = idx_sel.T.reshape(-1).astype(jnp.int32)
    wgh = jnp.take_along_axis(sig, idx_sel, axis=1)
    wv = wgh.T.reshape(-1).astype(jnp.float32)

    q = jnp.arange(MROW)
    samegrp = (q[:, None] // C) == (q[None, :] // C)
    upper = q[:, None] < q[None, :]
    bmask = (samegrp & upper).astype(jnp.float32) / (hw - 1)
    ws = wv.reshape(nstack, MROW)
    mask_w = ws[:, :, None] * ws[:, None, :] * bmask[None]

    x4 = x.reshape(B, ch, hw // 128, 128)
    nh = x4.shape[2]

    in_specs = [
        pl.BlockSpec((1, 1, nh, 128), functools.partial(_x_imap, i))
        for i in range(NCH)
    ] + [pl.BlockSpec((1, MROW, MROW), _sd_imap)]

    out = pl.pallas_call(
        _saw_body,
        grid_spec=pltpu.PrefetchScalarGridSpec(
            num_scalar_prefetch=1,
            grid=(B, TSTEPS),
            in_specs=in_specs,
            out_specs=pl.BlockSpec((1, 1, C, 1), _out_imap),
            scratch_shapes=[
                pltpu.VMEM((2, KSTEPS, SSTR * nh, 128), jnp.float32),
                pltpu.VMEM((MROW, MROW), jnp.float32),
            ],
        ),
        out_shape=jax.ShapeDtypeStruct((B, nstack, C, 1), jnp.float32),
        compiler_params=pltpu.CompilerParams(
            dimension_semantics=("parallel", "arbitrary"),
        ),
        name="saw_loss_mxu",
    )(chan, *([x4] * NCH), mask_w)
    total = jnp.sum(out) / B
    return total.reshape(1)


# speedup vs baseline: 1.1727x; 1.1727x over previous
import functools
import jax
import jax.numpy as jnp
from jax.experimental import pallas as pl
from jax.experimental.pallas import tpu as pltpu

C = 16
RELAX_DENOM = 2.0
NUM_OFF = C * (C - 1) / 2.0
MARGIN = float(int(NUM_OFF // RELAX_DENOM))

NCH = 128
MROW = 256
KSTEPS = MROW // NCH
SSTR = NCH + 1
CORES = 2
NSTACK = 2


def _x_imap(i, spc, c, t, chan_ref):
    tf = jnp.minimum(t, spc * NSTACK * KSTEPS - 1)
    gs = tf // KSTEPS
    return (c * spc + gs // NSTACK,
            chan_ref[(gs % NSTACK) * MROW + (tf % KSTEPS) * NCH + i], 0, 0)


def _sd_imap(spc, c, t, chan_ref):
    jd = jnp.clip((t - KSTEPS) // KSTEPS, 0, spc * NSTACK - 1)
    return (jd % NSTACK, 0, 0)


def _out_imap(spc, c, t, chan_ref):
    jd = jnp.clip((t - KSTEPS) // KSTEPS, 0, spc * NSTACK - 1)
    return (c * spc + jd // NSTACK, jd % NSTACK, 0, 0)


def _saw_body(chan_ref, *refs):
    xrefs = refs[:NCH]
    mask_ref = refs[NCH]
    out_ref = refs[NCH + 1]
    tile = refs[NCH + 2]
    acc = refs[NCH + 3]
    t = pl.program_id(1)
    fetch_t = pl.num_programs(1) - KSTEPS
    nh = xrefs[0].shape[2]

    @pl.when(t < fetch_t)
    def _():
        buf = (t // KSTEPS) % 2
        tk = tile.at[buf, t % KSTEPS]
        for i in range(NCH):
            tk[i : i + SSTR * nh : SSTR, :] = xrefs[i][0, 0]

    @pl.when(t >= KSTEPS)
    def _():
        jt = t - KSTEPS
        dbuf = (jt // KSTEPS) % 2
        ndots = nh // 2
        per = ndots // KSTEPS

        def dot_phase(p):
            @pl.when(jt % KSTEPS == p)
            def _():
                covs = []
                for j in range(p * per, (p + 1) * per):
                    lhs = jnp.concatenate(
                        [
                            jnp.concatenate(
                                [tile[dbuf, kk, jj * SSTR : jj * SSTR + NCH, :]
                                 for kk in range(KSTEPS)],
                                axis=0,
                            )
                            for jj in (2 * j, 2 * j + 1)
                        ],
                        axis=1,
                    )
                    covs.append(
                        jax.lax.dot_general(
                            lhs, lhs.T, (((1,), (0,)), ((), ())),
                            preferred_element_type=jnp.float32,
                        )
                    )
                while len(covs) > 1:
                    covs = [a + b for a, b in zip(covs[::2], covs[1::2])] + (
                        [covs[-1]] if len(covs) % 2 else [])
                part = covs[0]
                if p == 0 and KSTEPS > 1:
                    acc[...] = part
                if p == KSTEPS - 1:
                    cov = part if KSTEPS == 1 else acc[...] + part
                    tm = jnp.abs(cov) * mask_ref[0]
                    rs = jnp.sum(tm, axis=1, keepdims=True)
                    gs = jnp.sum(rs.reshape(C, C, 1), axis=1)
                    out_ref[0, 0] = jnp.maximum((gs - MARGIN) / NUM_OFF, 0.0)

        for p in range(KSTEPS):
            dot_phase(p)


def kernel(x, classifier_weight):
    B, ch, H, W = x.shape
    G = ch // C
    nstack = ch // MROW
    spc = B // CORES
    hw = H * W
    w = jnp.abs(classifier_weight)
    idx = jnp.argsort(-w, axis=1)
    idx_sel = idx[:C, :G]
    sig = jax.nn.sigmoid(w)[:C]
    chan = idx_sel.T.reshape(-1).astype(jnp.int32)
    wgh = jnp.take_along_axis(sig, idx_sel, axis=1)
    wv = wgh.T.reshape(-1).astype(jnp.float32)

    q = jnp.arange(MROW)
    samegrp = (q[:, None] // C) == (q[None, :] // C)
    upper = q[:, None] < q[None, :]
    bmask = (samegrp & upper).astype(jnp.float32) / (hw - 1)
    ws = wv.reshape(nstack, MROW)
    mask_w = ws[:, :, None] * ws[:, None, :] * bmask[None]

    x4 = x.reshape(B, ch, hw // 128, 128)
    nh = x4.shape[2]
    tsteps = spc * nstack * KSTEPS + KSTEPS

    in_specs = [
        pl.BlockSpec((1, 1, nh, 128), functools.partial(_x_imap, i, spc))
        for i in range(NCH)
    ] + [pl.BlockSpec((1, MROW, MROW), functools.partial(_sd_imap, spc))]

    out = pl.pallas_call(
        _saw_body,
        grid_spec=pltpu.PrefetchScalarGridSpec(
            num_scalar_prefetch=1,
            grid=(CORES, tsteps),
            in_specs=in_specs,
            out_specs=pl.BlockSpec((1, 1, C, 1), functools.partial(_out_imap, spc)),
            scratch_shapes=[
                pltpu.VMEM((2, KSTEPS, SSTR * nh, 128), jnp.float32),
                pltpu.VMEM((MROW, MROW), jnp.float32),
            ],
        ),
        out_shape=jax.ShapeDtypeStruct((B, nstack, C, 1), jnp.float32),
        compiler_params=pltpu.CompilerParams(
            dimension_semantics=("parallel", "arbitrary"),
        ),
        name="saw_loss_mxu",
    )(chan, *([x4] * NCH), mask_w)
    total = jnp.sum(out) / B
    return total.reshape(1)


# speedup vs baseline: 1.1756x; 1.0025x over previous
import functools
import jax
import jax.numpy as jnp
from jax.experimental import pallas as pl
from jax.experimental.pallas import tpu as pltpu

C = 16
RELAX_DENOM = 2.0
NUM_OFF = C * (C - 1) / 2.0
MARGIN = float(int(NUM_OFF // RELAX_DENOM))

NCH = 256
MROW = 256
KSTEPS = MROW // NCH
SSTR = NCH + 1
TILE_DT = jnp.float32


def _x_imap(i, b, s, k, chan_ref):
    return (b, chan_ref[s * MROW + k * NCH + i], 0, 0)


def _saw_body(chan_ref, *refs):
    xrefs = refs[:NCH]
    mask_ref = refs[NCH]
    out_ref = refs[NCH + 1]
    tile = refs[NCH + 2]
    k = pl.program_id(2)
    nh = xrefs[0].shape[2]
    tk = tile.at[k]
    for i in range(NCH):
        tk[i : i + SSTR * nh : SSTR, :] = xrefs[i][0, 0]

    @pl.when(k == KSTEPS - 1)
    def _():
        covs = []
        for j in range(nh // 2):
            lhs = jnp.concatenate(
                [
                    jnp.concatenate(
                        [tile[kk, jj * SSTR : jj * SSTR + NCH, :]
                         for kk in range(KSTEPS)],
                        axis=0,
                    )
                    for jj in (2 * j, 2 * j + 1)
                ],
                axis=1,
            )
            covs.append(
                jax.lax.dot_general(
                    lhs, lhs.T, (((1,), (0,)), ((), ())),
                    preferred_element_type=jnp.float32,
                )
            )
        while len(covs) > 1:
            covs = [a + b for a, b in zip(covs[::2], covs[1::2])] + (
                [covs[-1]] if len(covs) % 2 else [])
        t = jnp.abs(covs[0]) * mask_ref[0]
        rs = jnp.sum(t, axis=1, keepdims=True)
        gs = jnp.sum(rs.reshape(C, C, 1), axis=1)
        out_ref[0, 0] = jnp.maximum((gs - MARGIN) / NUM_OFF, 0.0)


def kernel(x, classifier_weight):
    B, ch, H, W = x.shape
    G = ch // C
    nstack = ch // MROW
    hw = H * W
    w = jnp.abs(classifier_weight)
    idx = jnp.argsort(-w, axis=1)
    idx_sel = idx[:C, :G]
    sig = jax.nn.sigmoid(w)[:C]
    chan = idx_sel.T.reshape(-1).astype(jnp.int32)
    wgh = jnp.take_along_axis(sig, idx_sel, axis=1)
    wv = wgh.T.reshape(-1).astype(jnp.float32)

    q = jnp.arange(MROW)
    samegrp = (q[:, None] // C) == (q[None, :] // C)
    upper = q[:, None] < q[None, :]
    bmask = (samegrp & upper).astype(jnp.float32) / (hw - 1)
    ws = wv.reshape(nstack, MROW)
    mask_w = ws[:, :, None] * ws[:, None, :] * bmask[None]

    x4 = x.reshape(B, ch, hw // 128, 128)
    nh = x4.shape[2]

    in_specs = [
        pl.BlockSpec((1, 1, nh, 128), functools.partial(_x_imap, i))
        for i in range(NCH)
    ] + [pl.BlockSpec((1, MROW, MROW), lambda b, s, k, c_r: (s, 0, 0))]

    out = pl.pallas_call(
        _saw_body,
        grid_spec=pltpu.PrefetchScalarGridSpec(
            num_scalar_prefetch=1,
            grid=(B, nstack, KSTEPS),
            in_specs=in_specs,
            out_specs=pl.BlockSpec((1, 1, C, 1), lambda b, s, k, c_r: (b, s, 0, 0)),
            scratch_shapes=[pltpu.VMEM((KSTEPS, SSTR * nh, 128), TILE_DT)],
        ),
        out_shape=jax.ShapeDtypeStruct((B, nstack, C, 1), jnp.float32),
        compiler_params=pltpu.CompilerParams(
            dimension_semantics=("parallel", "arbitrary", "arbitrary"),
        ),
        name="saw_loss_mxu",
    )(chan, *([x4] * NCH), mask_w)
    total = jnp.sum(out) / B
    return total.reshape(1)


# speedup vs baseline: 1.3130x; 1.1168x over previous
import functools
import jax
import jax.numpy as jnp
from jax.experimental import pallas as pl
from jax.experimental.pallas import tpu as pltpu

C = 16
RELAX_DENOM = 2.0
NUM_OFF = C * (C - 1) / 2.0
MARGIN = float(int(NUM_OFF // RELAX_DENOM))

NCH = 128
MROW = 256
KSTEPS = MROW // NCH
SSTR = NCH + 1
TILE_DT = jnp.float32


def _x_imap(i, b, s, k, chan_ref):
    return (b, chan_ref[s * MROW + k * NCH + i], 0, 0)


def _saw_body(chan_ref, *refs):
    xrefs = refs[:NCH]
    mask_ref = refs[NCH]
    out_ref = refs[NCH + 1]
    tile = refs[NCH + 2]
    k = pl.program_id(2)
    nh = xrefs[0].shape[2]
    tk = tile.at[k]
    for i in range(NCH):
        tk[i : i + SSTR * nh : SSTR, :] = xrefs[i][0, 0]

    @pl.when(k == KSTEPS - 1)
    def _():
        covs = []
        for j in range(nh // 2):
            lhs = jnp.concatenate(
                [
                    jnp.concatenate(
                        [tile[kk, jj * SSTR : jj * SSTR + NCH, :]
                         for kk in range(KSTEPS)],
                        axis=0,
                    )
                    for jj in (2 * j, 2 * j + 1)
                ],
                axis=1,
            )
            covs.append(
                jax.lax.dot_general(
                    lhs, lhs.T, (((1,), (0,)), ((), ())),
                    preferred_element_type=jnp.float32,
                )
            )
        while len(covs) > 1:
            covs = [a + b for a, b in zip(covs[::2], covs[1::2])] + (
                [covs[-1]] if len(covs) % 2 else [])
        t = jnp.abs(covs[0]) * mask_ref[0]
        rs = jnp.sum(t, axis=1, keepdims=True)
        gs = jnp.sum(rs.reshape(C, C, 1), axis=1)
        out_ref[0, 0] = jnp.maximum((gs - MARGIN) / NUM_OFF, 0.0)


def kernel(x, classifier_weight):
    B, ch, H, W = x.shape
    G = ch // C
    nstack = ch // MROW
    hw = H * W
    w = jnp.abs(classifier_weight)
    idx = jnp.argsort(-w, axis=1)
    idx_sel = idx[:C, :G]
    sig = jax.nn.sigmoid(w)[:C]
    chan = idx_sel.T.reshape(-1).astype(jnp.int32)
    wgh = jnp.take_along_axis(sig, idx_sel, axis=1)
    wv = wgh.T.reshape(-1).astype(jnp.float32)

    q = jnp.arange(MROW)
    samegrp = (q[:, None] // C) == (q[None, :] // C)
    upper = q[:, None] < q[None, :]
    bmask = (samegrp & upper).astype(jnp.float32) / (hw - 1)
    ws = wv.reshape(nstack, MROW)
    mask_w = ws[:, :, None] * ws[:, None, :] * bmask[None]

    x4 = x.reshape(B, ch, hw // 128, 128)
    nh = x4.shape[2]

    in_specs = [
        pl.BlockSpec((1, 1, nh, 128), functools.partial(_x_imap, i))
        for i in range(NCH)
    ] + [pl.BlockSpec((1, MROW, MROW), lambda b, s, k, c_r: (s, 0, 0))]

    out = pl.pallas_call(
        _saw_body,
        grid_spec=pltpu.PrefetchScalarGridSpec(
            num_scalar_prefetch=1,
            grid=(B, nstack, KSTEPS),
            in_specs=in_specs,
            out_specs=pl.BlockSpec((1, 1, C, 1), lambda b, s, k, c_r: (b, s, 0, 0)),
            scratch_shapes=[pltpu.VMEM((KSTEPS, SSTR * nh, 128), TILE_DT)],
        ),
        out_shape=jax.ShapeDtypeStruct((B, nstack, C, 1), jnp.float32),
        compiler_params=pltpu.CompilerParams(
            dimension_semantics=("parallel", "arbitrary", "arbitrary"),
        ),
        name="saw_loss_mxu",
    )(chan, *([x4] * NCH), mask_w)
    total = jnp.sum(out) / B
    return total.reshape(1)
